# baseline (speedup 1.0000x reference)
import jax
import jax.numpy as jnp
import numpy as np
from jax import lax
from jax.experimental import pallas as pl
from jax.experimental.pallas import tpu as pltpu
from jax.experimental.pallas import tpu_sc as plsc

N = 1_200_000
NW = 32
W = 37_504
C = 4_608
SIZES = [C] * (W // C) + [W - (W // C) * C]

_VS = np.array([0.05, 0.05, 0.1], dtype=np.float32)
RX, RY, RZ = 0.0, -40.0, -3.0
IVX = float(1.0 / np.float64(_VS[0]))
IVY = float(1.0 / np.float64(_VS[1]))
IVZ = float(1.0 / np.float64(_VS[2]))
GX, GY, GZ = 1408, 1600, 40


def _sc_body(pts_hbm, out_hbm, p0, p1, o0, o1, si0, si1, so0, so1):
    cid = lax.axis_index("c")
    sid = lax.axis_index("s")
    wid = sid * 2 + cid
    base = jnp.minimum(wid * W, N - W)
    base = pl.multiple_of(base, 128)

    pbufs = (p0, p1)
    obufs = (o0, o1)
    isems = (si0, si1)
    osems = (so0, so1)

    iota = lax.iota(jnp.int32, 16)
    col0 = iota * 0
    col1 = col0 + 1
    col2 = col0 + 2

    def compute(pb, ob, npts):
        def body(g, carry):
            rows = iota + g * 16
            vx = plsc.load_gather(pb, [rows, col0])
            vy = plsc.load_gather(pb, [rows, col1])
            vz = plsc.load_gather(pb, [rows, col2])
            tx = (vx - RX) * IVX
            ty = (vy - RY) * IVY
            tz = (vz - RZ) * IVZ
            cx = tx.astype(jnp.int32)
            cy = ty.astype(jnp.int32)
            cz = tz.astype(jnp.int32)
            ok = ((tx >= 0.0) & (cx < GX)
                  & (ty >= 0.0) & (cy < GY)
                  & (tz >= 0.0) & (cz < GZ))
            s = g * 16
            ob[0, pl.ds(s, 16)] = jnp.where(ok, cz, -1)
            ob[1, pl.ds(s, 16)] = jnp.where(ok, cy, -1)
            ob[2, pl.ds(s, 16)] = jnp.where(ok, cx, -1)
            return carry
        lax.fori_loop(0, npts // 16, body, 0)

    def in_copy(k, b):
        sz = SIZES[k]
        return pltpu.async_copy(
            pts_hbm.at[pl.ds(base + k * C, sz)],
            pbufs[b].at[pl.ds(0, sz)], isems[b])

    in_d = [None, None]
    out_d = [None, None]
    in_d[0] = in_copy(0, 0)
    off = 0
    for k, sz in enumerate(SIZES):
        b = k & 1
        if out_d[b] is not None:
            out_d[b].wait()
        in_d[b].wait()
        if k + 1 < len(SIZES):
            in_d[1 - b] = in_copy(k + 1, 1 - b)
        compute(pbufs[b], obufs[b], sz)
        out_d[b] = pltpu.async_copy(
            obufs[b].at[:, pl.ds(0, sz)],
            out_hbm.at[:, pl.ds(base + off, sz)], osems[b])
        off += sz
    for b in (0, 1):
        if out_d[b] is not None:
            out_d[b].wait()


_sc_call = pl.kernel(
    _sc_body,
    out_type=jax.ShapeDtypeStruct((3, N), jnp.int32),
    mesh=plsc.VectorSubcoreMesh(core_axis_name="c", subcore_axis_name="s"),
    compiler_params=pltpu.CompilerParams(needs_layout_passes=False, use_tc_tiling_on_sc=False),
    scratch_types=[
        pltpu.VMEM((C, 4), jnp.float32),
        pltpu.VMEM((C, 4), jnp.float32),
        pltpu.VMEM((3, C), jnp.int32),
        pltpu.VMEM((3, C), jnp.int32),
        pltpu.SemaphoreType.DMA,
        pltpu.SemaphoreType.DMA,
        pltpu.SemaphoreType.DMA,
        pltpu.SemaphoreType.DMA,
    ],
)


def kernel(input):
    return _sc_call(input)


# speedup vs baseline: 51.5127x; 51.5127x over previous
import jax
import jax.numpy as jnp
import numpy as np
from jax import lax
from jax.experimental import pallas as pl
from jax.experimental.pallas import tpu as pltpu
from jax.experimental.pallas import tpu_sc as plsc

N = 1_200_000
NW = 32
W = 37_504
C = 3_840
SIZES = [C] * (W // C) + [W - (W // C) * C]

_VS = np.array([0.05, 0.05, 0.1], dtype=np.float32)
RX, RY, RZ = 0.0, -40.0, -3.0
IVX = float(1.0 / np.float64(_VS[0]))
IVY = float(1.0 / np.float64(_VS[1]))
IVZ = float(1.0 / np.float64(_VS[2]))
GX, GY, GZ = 1408, 1600, 40


def _sc_body(pts_hbm, out_hbm, p0, p1, o0, o1, si0, si1, so0, so1):
    cid = lax.axis_index("c")
    sid = lax.axis_index("s")
    wid = sid * 2 + cid
    base = jnp.minimum(wid * W, N - W)
    base = pl.multiple_of(base, 128)

    pbufs = (p0, p1)
    obufs = (o0, o1)
    isems = (si0, si1)
    osems = (so0, so1)

    def compute(pb, ob, npts):
        def body(g, carry):
            s = g * 16
            vx = pb[0, pl.ds(s, 16)]
            vy = pb[1, pl.ds(s, 16)]
            vz = pb[2, pl.ds(s, 16)]
            tx = (vx - RX) * IVX
            ty = (vy - RY) * IVY
            tz = (vz - RZ) * IVZ
            cx = tx.astype(jnp.int32)
            cy = ty.astype(jnp.int32)
            cz = tz.astype(jnp.int32)
            ok = ((tx >= 0.0) & (cx < GX)
                  & (ty >= 0.0) & (cy < GY)
                  & (tz >= 0.0) & (cz < GZ))
            ob[0, pl.ds(s, 16)] = jnp.where(ok, cz, -1)
            ob[1, pl.ds(s, 16)] = jnp.where(ok, cy, -1)
            ob[2, pl.ds(s, 16)] = jnp.where(ok, cx, -1)
            return carry
        lax.fori_loop(0, npts // 16, body, 0)

    def in_copy(k, b):
        sz = SIZES[k]
        return pltpu.async_copy(
            pts_hbm.at[:, pl.ds(base + k * C, sz)],
            pbufs[b].at[:, pl.ds(0, sz)], isems[b])

    in_d = [None, None]
    out_d = [None, None]
    in_d[0] = in_copy(0, 0)
    off = 0
    for k, sz in enumerate(SIZES):
        b = k & 1
        if out_d[b] is not None:
            out_d[b].wait()
        in_d[b].wait()
        if k + 1 < len(SIZES):
            in_d[1 - b] = in_copy(k + 1, 1 - b)
        compute(pbufs[b], obufs[b], sz)
        out_d[b] = pltpu.async_copy(
            obufs[b].at[:, pl.ds(0, sz)],
            out_hbm.at[:, pl.ds(base + off, sz)], osems[b])
        off += sz
    for b in (0, 1):
        if out_d[b] is not None:
            out_d[b].wait()


_sc_call = pl.kernel(
    _sc_body,
    out_type=jax.ShapeDtypeStruct((3, N), jnp.int32),
    mesh=plsc.VectorSubcoreMesh(core_axis_name="c", subcore_axis_name="s"),
    compiler_params=pltpu.CompilerParams(needs_layout_passes=False),
    scratch_types=[
        pltpu.VMEM((4, C), jnp.float32),
        pltpu.VMEM((4, C), jnp.float32),
        pltpu.VMEM((3, C), jnp.int32),
        pltpu.VMEM((3, C), jnp.int32),
        pltpu.SemaphoreType.DMA,
        pltpu.SemaphoreType.DMA,
        pltpu.SemaphoreType.DMA,
        pltpu.SemaphoreType.DMA,
    ],
)


def kernel(input):
    return _sc_call(input.T)
